# Initial kernel scaffold; baseline (speedup 1.0000x reference)
import jax
import jax.numpy as jnp
from jax.experimental import pallas as pl


def kernel(x, edge_index):
    raise NotImplementedError("write your pallas kernel here")


# speedup vs baseline: 1.1013x; 1.1013x over previous
import functools

import jax
import jax.numpy as jnp
from jax import lax
from jax.experimental import pallas as pl
from jax.experimental.pallas import tpu as pltpu, tpu_sc as plsc

_NC = 2
_NS = 16
_NW = _NC * _NS
_D = 128
_C = 80


def _dot_chunk(srows, drows, outv):
    lanes = lax.iota(jnp.int32, 16)

    def group_body(g, _):
        e0 = g * 16
        rows = lanes + e0

        def d_body(d, acc):
            col = jnp.full((16,), d, jnp.int32)
            sv = plsc.load_gather(srows, [rows, col])
            dv = plsc.load_gather(drows, [rows, col])
            return acc + sv * dv

        acc = lax.fori_loop(0, _D, d_body, jnp.zeros((16,), jnp.float32),
                            unroll=8)
        outv[pl.ds(e0, 16)] = acc
        return 0

    lax.fori_loop(0, _C // 16, group_body, 0)


def _make_sc_kernel(E):
    assert E % (_NW * _C) == 0
    epw = E // _NW
    n_chunks = epw // _C
    mesh = plsc.VectorSubcoreMesh(
        core_axis_name="c", subcore_axis_name="s",
        num_cores=_NC, num_subcores=_NS)

    @functools.partial(
        pl.kernel,
        out_type=jax.ShapeDtypeStruct((E,), jnp.float32),
        mesh=mesh,
        compiler_params=pltpu.CompilerParams(needs_layout_passes=False),
        scratch_types=[
            pltpu.VMEM((_C,), jnp.int32),
            pltpu.VMEM((_C,), jnp.int32),
            pltpu.VMEM((_C, _D), jnp.float32),
            pltpu.VMEM((_C, _D), jnp.float32),
            pltpu.VMEM((_C,), jnp.float32),
            pltpu.SemaphoreType.DMA,
            pltpu.SemaphoreType.DMA,
        ],
    )
    def sc_kernel(x_hbm, src_hbm, dst_hbm, out_hbm,
                  sidx, didx, srows, drows, outv, sem_s, sem_d):
        wid = lax.axis_index("s") * _NC + lax.axis_index("c")
        base = wid * epw

        def chunk_body(ci, _):
            off = base + ci * _C
            pltpu.sync_copy(src_hbm.at[pl.ds(off, _C)], sidx)
            pltpu.sync_copy(dst_hbm.at[pl.ds(off, _C)], didx)
            cp_s = pltpu.async_copy(x_hbm.at[sidx], srows, sem_s)
            cp_d = pltpu.async_copy(x_hbm.at[didx], drows, sem_d)
            cp_s.wait()
            cp_d.wait()
            _dot_chunk(srows, drows, outv)
            pltpu.sync_copy(outv, out_hbm.at[pl.ds(off, _C)])
            return 0

        lax.fori_loop(0, n_chunks, chunk_body, 0)

    return sc_kernel


def kernel(x, edge_index):
    src = edge_index[0].astype(jnp.int32)
    dst = edge_index[1].astype(jnp.int32)
    e = src.shape[0]
    e_pad = ((e + _NW * _C - 1) // (_NW * _C)) * (_NW * _C)
    if e_pad != e:
        pad = jnp.zeros((e_pad - e,), jnp.int32)
        src = jnp.concatenate([src, pad])
        dst = jnp.concatenate([dst, pad])
    out = _make_sc_kernel(e_pad)(x, src, dst)
    return out[:e]


# speedup vs baseline: 1.3073x; 1.1870x over previous
import functools

import jax
import jax.numpy as jnp
from jax import lax
from jax.experimental import pallas as pl
from jax.experimental.pallas import tpu as pltpu, tpu_sc as plsc

_NC = 2
_NS = 16
_NW = _NC * _NS
_D = 128
_C = 128


def _dot_chunk(sbuf, dbuf, outv, lidx):
    lanes = lax.iota(jnp.int32, 16)

    def group_body(g, _):
        rows = lanes + g * 16

        def d_body(d, carry):
            acc, col = carry
            sv = plsc.load_gather(sbuf, [rows, col])
            dv = plsc.load_gather(dbuf, [rows, col])
            return acc + sv * dv, col + 1

        acc, _ = lax.fori_loop(
            0, _D, d_body,
            (jnp.zeros((16,), jnp.float32), jnp.zeros((16,), jnp.int32)),
            unroll=32)
        outv[pl.ds(lidx + g * 16, 16)] = acc
        return 0

    lax.fori_loop(0, _C // 16, group_body, 0)


def _make_sc_kernel(E):
    assert E % (_NW * _C) == 0
    epw = E // _NW
    n_chunks = epw // _C
    mesh = plsc.VectorSubcoreMesh(
        core_axis_name="c", subcore_axis_name="s",
        num_cores=_NC, num_subcores=_NS)

    @functools.partial(
        pl.kernel,
        out_type=jax.ShapeDtypeStruct((E,), jnp.float32),
        mesh=mesh,
        compiler_params=pltpu.CompilerParams(needs_layout_passes=False),
        scratch_types=[
            pltpu.VMEM((epw,), jnp.int32),
            pltpu.VMEM((epw,), jnp.int32),
            pltpu.VMEM((epw,), jnp.float32),
            pltpu.VMEM((_C, _D), jnp.float32),
            pltpu.VMEM((_C, _D), jnp.float32),
            pltpu.VMEM((_C, _D), jnp.float32),
            pltpu.VMEM((_C, _D), jnp.float32),
            pltpu.SemaphoreType.DMA,
            pltpu.SemaphoreType.DMA,
        ],
    )
    def sc_kernel(x_hbm, src_hbm, dst_hbm, out_hbm,
                  sidx, didx, outv, sbuf0, dbuf0, sbuf1, dbuf1,
                  sem0, sem1):
        wid = lax.axis_index("s") * _NC + lax.axis_index("c")
        base = wid * epw

        pltpu.sync_copy(src_hbm.at[pl.ds(base, epw)], sidx)
        pltpu.sync_copy(dst_hbm.at[pl.ds(base, epw)], didx)

        def start(ci, sbuf, dbuf, sem):
            lidx = ci * _C
            pltpu.async_copy(x_hbm.at[sidx.at[pl.ds(lidx, _C)]], sbuf, sem)
            pltpu.async_copy(x_hbm.at[didx.at[pl.ds(lidx, _C)]], dbuf, sem)

        def drain(sbuf, dbuf, sem):
            pltpu.make_async_copy(x_hbm.at[sidx.at[pl.ds(0, _C)]],
                                  sbuf, sem).wait()
            pltpu.make_async_copy(x_hbm.at[didx.at[pl.ds(0, _C)]],
                                  dbuf, sem).wait()

        start(0, sbuf0, dbuf0, sem0)
        start(1, sbuf1, dbuf1, sem1)

        def pair_body(p, _):
            ca = 2 * p
            drain(sbuf0, dbuf0, sem0)
            _dot_chunk(sbuf0, dbuf0, outv, ca * _C)

            @pl.when(ca + 2 < n_chunks)
            def _():
                start(ca + 2, sbuf0, dbuf0, sem0)

            drain(sbuf1, dbuf1, sem1)
            _dot_chunk(sbuf1, dbuf1, outv, (ca + 1) * _C)

            @pl.when(ca + 3 < n_chunks)
            def _():
                start(ca + 3, sbuf1, dbuf1, sem1)

            return 0

        lax.fori_loop(0, n_chunks // 2, pair_body, 0)
        pltpu.sync_copy(outv, out_hbm.at[pl.ds(base, epw)])

    return sc_kernel


def kernel(x, edge_index):
    src = edge_index[0].astype(jnp.int32)
    dst = edge_index[1].astype(jnp.int32)
    e = src.shape[0]
    quantum = _NW * _C * 2
    e_pad = ((e + quantum - 1) // quantum) * quantum
    if e_pad != e:
        pad = jnp.zeros((e_pad - e,), jnp.int32)
        src = jnp.concatenate([src, pad])
        dst = jnp.concatenate([dst, pad])
    out = _make_sc_kernel(e_pad)(x, src, dst)
    return out[:e]


# speedup vs baseline: 3.1561x; 2.4143x over previous
import functools

import jax
import jax.numpy as jnp
from jax import lax
from jax.experimental import pallas as pl
from jax.experimental.pallas import tpu as pltpu, tpu_sc as plsc

_NC = 2
_NS = 16
_NW = _NC * _NS
_D = 128
_C = 128


def _dot_chunk(sbuf, dbuf, outv, lidx):
    lane15 = lax.iota(jnp.int32, 16) == 15

    def edge_body(e, _):
        acc = jnp.zeros((16,), jnp.float32)
        for q in range(_D // 32):
            sq = plsc.bitcast(sbuf[e, pl.ds(16 * q, 16)], jnp.bfloat16)
            dq = plsc.bitcast(dbuf[e, pl.ds(16 * q, 16)], jnp.bfloat16)
            p = sq * dq
            a, b = plsc.unpack(p, format=plsc.PackFormat.INTERLEAVED)
            acc = acc + a
            acc = acc + b
        cum = plsc.cumsum(acc)
        plsc.store_scatter(outv, [jnp.full((16,), lidx + e, jnp.int32)],
                           cum, mask=lane15)
        return 0

    lax.fori_loop(0, _C, edge_body, 0, unroll=4)


def _make_sc_kernel(E):
    assert E % (_NW * _C) == 0
    epw = E // _NW
    n_chunks = epw // _C
    mesh = plsc.VectorSubcoreMesh(
        core_axis_name="c", subcore_axis_name="s",
        num_cores=_NC, num_subcores=_NS)

    @functools.partial(
        pl.kernel,
        out_type=jax.ShapeDtypeStruct((E,), jnp.float32),
        mesh=mesh,
        compiler_params=pltpu.CompilerParams(needs_layout_passes=False, use_tc_tiling_on_sc=False),
        scratch_types=[
            pltpu.VMEM((epw,), jnp.int32),
            pltpu.VMEM((epw,), jnp.int32),
            pltpu.VMEM((epw,), jnp.float32),
            pltpu.VMEM((_C, _D // 2), jnp.int32),
            pltpu.VMEM((_C, _D // 2), jnp.int32),
            pltpu.VMEM((_C, _D // 2), jnp.int32),
            pltpu.VMEM((_C, _D // 2), jnp.int32),
            pltpu.SemaphoreType.DMA,
            pltpu.SemaphoreType.DMA,
        ],
    )
    def sc_kernel(x_hbm, src_hbm, dst_hbm, out_hbm,
                  sidx, didx, outv, sbuf0, dbuf0, sbuf1, dbuf1,
                  sem0, sem1):
        wid = lax.axis_index("s") * _NC + lax.axis_index("c")
        base = wid * epw

        pltpu.sync_copy(src_hbm.at[pl.ds(base, epw)], sidx)
        pltpu.sync_copy(dst_hbm.at[pl.ds(base, epw)], didx)

        def start(ci, sbuf, dbuf, sem):
            lidx = ci * _C
            pltpu.async_copy(x_hbm.at[sidx.at[pl.ds(lidx, _C)]], sbuf, sem)
            pltpu.async_copy(x_hbm.at[didx.at[pl.ds(lidx, _C)]], dbuf, sem)

        def drain(sbuf, dbuf, sem):
            pltpu.make_async_copy(x_hbm.at[sidx.at[pl.ds(0, _C)]],
                                  sbuf, sem).wait()
            pltpu.make_async_copy(x_hbm.at[didx.at[pl.ds(0, _C)]],
                                  dbuf, sem).wait()

        start(0, sbuf0, dbuf0, sem0)
        start(1, sbuf1, dbuf1, sem1)

        def pair_body(p, _):
            ca = 2 * p
            drain(sbuf0, dbuf0, sem0)
            _dot_chunk(sbuf0, dbuf0, outv, ca * _C)

            @pl.when(ca + 2 < n_chunks)
            def _():
                start(ca + 2, sbuf0, dbuf0, sem0)

            drain(sbuf1, dbuf1, sem1)
            _dot_chunk(sbuf1, dbuf1, outv, (ca + 1) * _C)

            @pl.when(ca + 3 < n_chunks)
            def _():
                start(ca + 3, sbuf1, dbuf1, sem1)

            return 0

        lax.fori_loop(0, n_chunks // 2, pair_body, 0)
        pltpu.sync_copy(outv, out_hbm.at[pl.ds(base, epw)])

    return sc_kernel


def kernel(x, edge_index):
    xb = x.astype(jnp.bfloat16).reshape(x.shape[0], x.shape[1] // 2, 2)
    xb = jax.lax.bitcast_convert_type(xb, jnp.int32)
    src = edge_index[0].astype(jnp.int32)
    dst = edge_index[1].astype(jnp.int32)
    e = src.shape[0]
    quantum = _NW * _C * 2
    e_pad = ((e + quantum - 1) // quantum) * quantum
    if e_pad != e:
        pad = jnp.zeros((e_pad - e,), jnp.int32)
        src = jnp.concatenate([src, pad])
        dst = jnp.concatenate([dst, pad])
    out = _make_sc_kernel(e_pad)(xb, src, dst)
    return out[:e]
